# baseline (speedup 1.0000x reference)
import functools

import jax
import jax.numpy as jnp
from jax import lax
from jax.experimental import pallas as pl
from jax.experimental.pallas import tpu as pltpu
from jax.experimental.pallas import tpu_sc as plsc

HIDDEN = 64
NUM_LAYERS = 3
N_JOINT = 50000
N_BAR = 400000
E = 800000
NUM_GRAPHS = 16

NC = 2
NS = 16
NW = NC * NS
CHUNK = 25000
CHUNK_A = 25088
STRIPE = 1568
LAST_STRIPE = CHUNK - 15 * STRIPE
EP = E + 1024
DUMP = EP - 16
BB = 128
PER_TILE = E // NW


def _lane_i32(row, idx, lanes):
    return jnp.max(jnp.where(lanes == idx, row, jnp.int32(-2147483647)))


def _mesh():
    return plsc.VectorSubcoreMesh(core_axis_name="c", subcore_axis_name="s",
                                  num_cores=NC, num_subcores=NS)


def _make_count(nchunk):
    @functools.partial(
        pl.kernel,
        out_type=jax.ShapeDtypeStruct((NW, 16), jnp.int32),
        mesh=_mesh(),
        compiler_params=pltpu.CompilerParams(needs_layout_passes=False),
        scratch_types=[
            pltpu.VMEM((2048,), jnp.int32),
            pltpu.VMEM((16,), jnp.int32),
        ],
    )
    def k(dst_hbm, counts_hbm, dbuf, crow):
        c = lax.axis_index("c")
        s = lax.axis_index("s")
        wid = c * NS + s
        lanes = lax.iota(jnp.int32, 16)
        base0 = wid * PER_TILE

        def groups(ngroups, limit, counts):
            def g_body(g, counts):
                d16 = dbuf[pl.ds(g * 16, 16)]
                valid = (g * 16 + lanes) < limit
                bkt = jnp.where(valid, d16 // CHUNK, 0)

                def k_body(kk, counts):
                    mk = valid & (bkt == kk)
                    pc = plsc.all_reduce_population_count(mk)
                    return counts + jnp.where(lanes == kk, pc, 0)

                return lax.fori_loop(0, nchunk, k_body, counts)

            return lax.fori_loop(0, ngroups, g_body, counts)

        def r_body(r, counts):
            pltpu.sync_copy(dst_hbm.at[pl.ds(base0 + r * 2048, 2048)], dbuf)
            return groups(128, 2048, counts)

        counts = lax.fori_loop(0, 12, r_body, jnp.zeros((16,), jnp.int32))
        pltpu.sync_copy(dst_hbm.at[pl.ds(base0 + 12 * 2048, 424)],
                        dbuf.at[pl.ds(0, 424)])
        counts = groups(27, 424, counts)
        crow[...] = counts
        pltpu.sync_copy(crow, counts_hbm.at[wid])

    return k


def _make_place(nchunk):
    @functools.partial(
        pl.kernel,
        out_type=(
            jax.ShapeDtypeStruct((EP,), jnp.int32),
            jax.ShapeDtypeStruct((EP,), jnp.int32),
            jax.ShapeDtypeStruct((2, 16), jnp.int32),
        ),
        mesh=_mesh(),
        compiler_params=pltpu.CompilerParams(needs_layout_passes=False),
        scratch_types=[
            pltpu.VMEM((NW, 16), jnp.int32),
            pltpu.VMEM((16,), jnp.int32),
            pltpu.VMEM((128,), jnp.int32),
            pltpu.VMEM((128,), jnp.int32),
            pltpu.VMEM((128,), jnp.int32),
            pltpu.VMEM((128,), jnp.int32),
            pltpu.VMEM((128,), jnp.int32),
            pltpu.VMEM((2, 16), jnp.int32),
            pltpu.SemaphoreType.DMA,
        ],
    )
    def k(src_hbm, dst_hbm, counts_hbm, srcb, dstb, offs,
          cts, wb, ssrc, sdst, pos_st, src_st, dst_st, offs_st, sem):
        c = lax.axis_index("c")
        s = lax.axis_index("s")
        wid = c * NS + s
        lanes = lax.iota(jnp.int32, 16)
        pltpu.sync_copy(counts_hbm, cts)

        def tot_body(t, total):
            return total + cts[t, :]

        total = lax.fori_loop(0, NW, tot_body, jnp.zeros((16,), jnp.int32))
        pad8 = ((total + 7) // 8) * 8
        starts = plsc.cumsum(pad8) - pad8

        def pref_body(t, pref):
            tv = jnp.broadcast_to(t, (16,)).astype(jnp.int32)
            wv = jnp.broadcast_to(wid, (16,)).astype(jnp.int32)
            return pref + jnp.where(tv < wv, cts[t, :], 0)

        pref = lax.fori_loop(0, NW, pref_body, jnp.zeros((16,), jnp.int32))
        wb[...] = starts + pref

        @pl.when(wid == 0)
        def _():
            offs_st[0, :] = starts
            offs_st[1, :] = total
            pltpu.sync_copy(offs_st, offs)

        base0 = wid * PER_TILE

        def do_block(limit):
            def g_body(g, _):
                off = g * 16
                s16 = ssrc[pl.ds(off, 16)]
                d16 = sdst[pl.ds(off, 16)]
                valid = (off + lanes) < limit
                bkt = jnp.where(valid, d16 // CHUNK, 0)
                wbg = plsc.load_gather(wb, [bkt])

                def k_body(kk, carry):
                    rank, inc = carry
                    mk = valid & (bkt == kk)
                    csum = plsc.cumsum(mk.astype(jnp.int32))
                    rank = rank + jnp.where(mk, csum - 1, 0)
                    inc = inc + jnp.where(
                        lanes == kk, plsc.all_reduce_population_count(mk), 0)
                    return (rank, inc)

                rank, inc = lax.fori_loop(
                    0, nchunk, k_body,
                    (jnp.zeros((16,), jnp.int32), jnp.zeros((16,), jnp.int32)))
                pos = jnp.where(valid, wbg + rank, DUMP + lanes)
                wb[...] = wb[...] + inc
                pos_st[pl.ds(off, 16)] = pos
                src_st[pl.ds(off, 16)] = s16
                dst_st[pl.ds(off, 16)] = d16
                return 0

            lax.fori_loop(0, 8, g_body, 0)
            pltpu.async_copy(src_st, srcb.at[pos_st], sem).wait()
            pltpu.async_copy(dst_st, dstb.at[pos_st], sem).wait()

        def r_body(r, _):
            base = base0 + r * 128
            pltpu.sync_copy(src_hbm.at[pl.ds(base, 128)], ssrc)
            pltpu.sync_copy(dst_hbm.at[pl.ds(base, 128)], sdst)
            do_block(128)
            return 0

        lax.fori_loop(0, 195, r_body, 0)
        tb = base0 + 195 * 128
        pltpu.sync_copy(src_hbm.at[pl.ds(tb, 40)], ssrc.at[pl.ds(0, 40)])
        pltpu.sync_copy(dst_hbm.at[pl.ds(tb, 40)], sdst.at[pl.ds(0, 40)])
        do_block(40)

    return k


def _make_agg(nsrc, ndst, nchunk):
    chunks_per_sc = nchunk // NC

    @functools.partial(
        pl.kernel,
        out_type=(
            jax.ShapeDtypeStruct((ndst, HIDDEN), jnp.float32),
            jax.ShapeDtypeStruct((ndst,), jnp.float32),
        ),
        mesh=_mesh(),
        compiler_params=pltpu.CompilerParams(needs_layout_passes=False,
                                             use_tc_tiling_on_sc=False),
        scratch_types=[
            pltpu.VMEM_SHARED((CHUNK_A, HIDDEN), jnp.float32),
            pltpu.VMEM_SHARED((CHUNK_A,), jnp.float32),
            pltpu.VMEM((BB,), jnp.float32),
            pltpu.VMEM((BB,), jnp.int32),
            pltpu.VMEM((BB,), jnp.int32),
            pltpu.VMEM((BB,), jnp.int32),
            pltpu.VMEM((BB,), jnp.float32),
            pltpu.VMEM((BB,), jnp.float32),
            pltpu.VMEM((BB, HIDDEN), jnp.float32),
            pltpu.VMEM((STRIPE,), jnp.float32),
            pltpu.VMEM((2, 16), jnp.int32),
            pltpu.VMEM((16,), jnp.float32),
            pltpu.SemaphoreType.DMA,
        ],
    )
    def k(srcb, dstb, offs, a_s, a_d, xs, gv, acc_hbm, s_hbm,
          out_sp, s_sp, adv_b, src_v, dst_v, dstl_v, asv, w_v, rows,
          szbuf, offs_v, gv_v, sem):
        c = lax.axis_index("c")
        t = lax.axis_index("s")
        lanes = lax.iota(jnp.int32, 16)
        pltpu.sync_copy(offs, offs_v)
        pltpu.sync_copy(gv, gv_v)
        gvec = gv_v[...]

        def z1(i, _):
            szbuf[pl.ds(i * 16, 16)] = jnp.zeros((16,), jnp.float32)
            return 0

        lax.fori_loop(0, STRIPE // 16, z1, 0)

        def z2(q, _):
            rows[q // 4, pl.ds((q % 4) * 16, 16)] = jnp.zeros((16,), jnp.float32)
            return 0

        r0 = pl.multiple_of(t * STRIPE, 8)
        for cc in range(chunks_per_sc):
            ci = c * chunks_per_sc + cc
            lo = pl.multiple_of(ci * CHUNK, 8)
            start = pl.multiple_of(_lane_i32(offs_v[0, :], ci, lanes), 8)
            cnt = _lane_i32(offs_v[1, :], ci, lanes)
            lax.fori_loop(0, BB * 4, z2, 0)
            for kk in range(12):
                pltpu.sync_copy(rows, out_sp.at[pl.ds(r0 + kk * 128, 128)])
            pltpu.sync_copy(rows.at[pl.ds(0, 32)],
                            out_sp.at[pl.ds(r0 + 12 * 128, 32)])
            pltpu.sync_copy(szbuf, s_sp.at[pl.ds(r0, STRIPE)])
            plsc.subcore_barrier()

            share = pl.multiple_of(((cnt + 15) // 16 + 7) // 8 * 8, 8)
            t0 = start + t * share
            nb = (share + BB - 1) // BB
            mylim = jnp.minimum(cnt - t * share, share)

            def b_body(i, _):
                base = pl.multiple_of(t0 + i * BB, 8)
                lim = mylim - i * BB
                pltpu.sync_copy(srcb.at[pl.ds(base, BB)], src_v)
                pltpu.sync_copy(dstb.at[pl.ds(base, BB)], dst_v)

                def g_body(g, _):
                    off = g * 16
                    valid = (off + lanes) < lim
                    d16 = dst_v[pl.ds(off, 16)]
                    s16 = jnp.where(valid, src_v[pl.ds(off, 16)], 0)
                    dg = jnp.where(valid, d16, lo)
                    src_v[pl.ds(off, 16)] = s16
                    dst_v[pl.ds(off, 16)] = dg
                    dstl_v[pl.ds(off, 16)] = dg - lo
                    return 0

                lax.fori_loop(0, BB // 16, g_body, 0)
                cp_as = pltpu.async_copy(a_s.at[src_v], asv, sem)
                cp_ad = pltpu.async_copy(a_d.at[dst_v], adv_b, sem)
                cp_as.wait()
                cp_ad.wait()

                def g2(g, _):
                    off = g * 16
                    valid = (off + lanes) < lim
                    pre = asv[pl.ds(off, 16)] + adv_b[pl.ds(off, 16)]
                    e = jnp.where(pre >= 0, pre, 0.2 * pre)
                    w = jnp.where(valid, jnp.exp(e - gvec), 0.0)
                    w_v[pl.ds(off, 16)] = w
                    return 0

                lax.fori_loop(0, BB // 16, g2, 0)
                pltpu.sync_copy(w_v, s_sp.at[dstl_v], add=True)
                pltpu.async_copy(xs.at[src_v], rows, sem).wait()

                def wmul(g, _):
                    off = g * 16
                    w16 = w_v[pl.ds(off, 16)]
                    row_idx = off + lanes

                    def d_body(d, _):
                        col = jnp.broadcast_to(d, (16,)).astype(jnp.int32)
                        vals = plsc.load_gather(rows, [row_idx, col])
                        plsc.store_scatter(rows, [row_idx, col], vals * w16)
                        return 0

                    lax.fori_loop(0, HIDDEN, d_body, 0)
                    return 0

                lax.fori_loop(0, BB // 16, wmul, 0)
                pltpu.sync_copy(rows, out_sp.at[dstl_v], add=True)
                return 0

            lax.fori_loop(0, nb, b_body, 0)
            plsc.subcore_barrier()

            @pl.when(t < 15)
            def _():
                pltpu.sync_copy(out_sp.at[pl.ds(r0, STRIPE)],
                                acc_hbm.at[pl.ds(lo + r0, STRIPE)])
                pltpu.sync_copy(s_sp.at[pl.ds(r0, STRIPE)],
                                s_hbm.at[pl.ds(lo + r0, STRIPE)])

            @pl.when(t == 15)
            def _():
                pltpu.sync_copy(out_sp.at[pl.ds(r0, LAST_STRIPE)],
                                acc_hbm.at[pl.ds(lo + r0, LAST_STRIPE)])
                pltpu.sync_copy(s_sp.at[pl.ds(r0, LAST_STRIPE)],
                                s_hbm.at[pl.ds(lo + r0, LAST_STRIPE)])

    return k


_COUNT = {n: _make_count(n) for n in (2, 16)}
_PLACE = {n: _make_place(n) for n in (2, 16)}
_AGG_BJ = _make_agg(N_BAR, N_JOINT, 2)
_AGG_JB = _make_agg(N_JOINT, N_BAR, 16)

_NEG = -3.4e38


def _make_prep(n, blk, first, f_raw):
    grid = (n // blk,)

    def body(*refs):
        if first:
            (raw, w_enc, b_enc, w_a, att_a, w_b, att_b,
             xs_o, as_o, ad_o, ms_o, md_o) = refs
            x = raw[...] @ w_enc[...] + b_enc[...]
        else:
            (acc, s, bias, w_a, att_a, w_b, att_b,
             xs_o, as_o, ad_o, ms_o, md_o) = refs
            x = jax.nn.relu(acc[...] / (s[...] + 1e-16) + bias[...])
        xs = x @ w_a[...]
        a_s = jnp.sum(xs * att_a[...], axis=1, keepdims=True)
        v = w_b[...] @ att_b[...][:, None]
        a_d = x @ v
        xs_o[...] = xs
        as_o[...] = a_s
        ad_o[...] = a_d
        i = pl.program_id(0)

        @pl.when(i == 0)
        def _():
            ms_o[...] = jnp.full((1, 1), _NEG, jnp.float32)
            md_o[...] = jnp.full((1, 1), _NEG, jnp.float32)

        ms_o[...] = jnp.maximum(ms_o[...], jnp.max(a_s))
        md_o[...] = jnp.maximum(md_o[...], jnp.max(a_d))

    vec = pl.BlockSpec((HIDDEN,), lambda i: (0,))
    mat = pl.BlockSpec((HIDDEN, HIDDEN), lambda i: (0, 0))
    one = pl.BlockSpec((1, 1), lambda i: (0, 0))
    if first:
        in_specs = [pl.BlockSpec((blk, f_raw), lambda i: (i, 0)),
                    pl.BlockSpec((f_raw, HIDDEN), lambda i: (0, 0)), vec]
    else:
        in_specs = [pl.BlockSpec((blk, HIDDEN), lambda i: (i, 0)),
                    pl.BlockSpec((blk, 1), lambda i: (i, 0)), vec]
    in_specs += [mat, vec, mat, vec]
    return pl.pallas_call(
        body,
        grid=grid,
        in_specs=in_specs,
        out_specs=[
            pl.BlockSpec((blk, HIDDEN), lambda i: (i, 0)),
            pl.BlockSpec((blk, 1), lambda i: (i, 0)),
            pl.BlockSpec((blk, 1), lambda i: (i, 0)),
            one, one,
        ],
        out_shape=[
            jax.ShapeDtypeStruct((n, HIDDEN), jnp.float32),
            jax.ShapeDtypeStruct((n, 1), jnp.float32),
            jax.ShapeDtypeStruct((n, 1), jnp.float32),
            jax.ShapeDtypeStruct((1, 1), jnp.float32),
            jax.ShapeDtypeStruct((1, 1), jnp.float32),
        ],
    )


_PREP_J0 = _make_prep(N_JOINT, 2000, True, 6)
_PREP_B0 = _make_prep(N_BAR, 2000, True, 4)
_PREP_J = _make_prep(N_JOINT, 2000, False, 0)
_PREP_B = _make_prep(N_BAR, 2000, False, 0)


_HBLK = 2000


def _head_body(acc, s, bias, aw1, ab1, ag, abe, aw2, ab2, batch,
               cw1, cb1, cg, cbe, cw2, cb2,
               probs_o, value_o, pooled_sc, cnt_sc):
    i = pl.program_id(0)
    b = jax.nn.relu(acc[...] / (s[...] + 1e-16) + bias[...])
    h = b @ aw1[...] + ab1[...]
    m = jnp.mean(h, axis=-1, keepdims=True)
    v = jnp.mean((h - m) ** 2, axis=-1, keepdims=True)
    h = (h - m) / jnp.sqrt(v + 1e-5) * ag[...] + abe[...]
    h = jax.nn.relu(h)
    probs_o[...] = jax.nn.sigmoid(h @ aw2[...] + ab2[...])

    @pl.when(i == 0)
    def _():
        pooled_sc[...] = jnp.zeros_like(pooled_sc)
        cnt_sc[...] = jnp.zeros_like(cnt_sc)

    gids = lax.broadcasted_iota(jnp.int32, (NUM_GRAPHS, 1), 0)
    onehot = (batch[...][:, 0][None, :] == gids).astype(jnp.float32)
    pooled_sc[...] = pooled_sc[...] + onehot @ b
    cnt_sc[...] = cnt_sc[...] + jnp.sum(onehot, axis=1, keepdims=True)

    @pl.when(i == (N_BAR // _HBLK) - 1)
    def _():
        pooled = pooled_sc[...] / jnp.maximum(cnt_sc[...], 1.0)
        h2 = pooled @ cw1[...] + cb1[...]
        m2 = jnp.mean(h2, axis=-1, keepdims=True)
        v2 = jnp.mean((h2 - m2) ** 2, axis=-1, keepdims=True)
        h2 = (h2 - m2) / jnp.sqrt(v2 + 1e-5) * cg[...] + cbe[...]
        h2 = jax.nn.relu(h2)
        value_o[...] = jnp.tanh(h2 @ cw2[...] + cb2[...])[:, 0]


def _make_head():
    vec = pl.BlockSpec((HIDDEN,), lambda i: (0,))
    mat = pl.BlockSpec((HIDDEN, HIDDEN), lambda i: (0, 0))
    col = pl.BlockSpec((HIDDEN, 1), lambda i: (0, 0))
    one = pl.BlockSpec((1,), lambda i: (0,))
    return pl.pallas_call(
        _head_body,
        grid=(N_BAR // _HBLK,),
        in_specs=[
            pl.BlockSpec((_HBLK, HIDDEN), lambda i: (i, 0)),
            pl.BlockSpec((_HBLK, 1), lambda i: (i, 0)),
            vec,
            mat, vec, vec, vec, col, one,
            pl.BlockSpec((_HBLK, 1), lambda i: (i, 0)),
            mat, vec, vec, vec, col, one,
        ],
        out_specs=[
            pl.BlockSpec((_HBLK, 1), lambda i: (i, 0)),
            pl.BlockSpec((NUM_GRAPHS,), lambda i: (0,)),
        ],
        out_shape=[
            jax.ShapeDtypeStruct((N_BAR, 1), jnp.float32),
            jax.ShapeDtypeStruct((NUM_GRAPHS,), jnp.float32),
        ],
        scratch_shapes=[
            pltpu.VMEM((NUM_GRAPHS, HIDDEN), jnp.float32),
            pltpu.VMEM((NUM_GRAPHS, 1), jnp.float32),
        ],
    )


_HEAD = _make_head()


def _gvec(ms, md):
    g = jax.nn.leaky_relu(ms[0, 0] + md[0, 0], negative_slope=0.2)
    return jnp.full((16,), g, jnp.float32)


def kernel(joint_x, bar_x, params, bj_src, bj_dst, jb_src, jb_dst, bar_batch):
    p = params
    cnt_bj = _COUNT[2](bj_dst)
    srcb_bj, dstb_bj, offs_bj = _PLACE[2](bj_src, bj_dst, cnt_bj)
    cnt_jb = _COUNT[16](jb_dst)
    srcb_jb, dstb_jb, offs_jb = _PLACE[16](jb_src, jb_dst, cnt_jb)

    acc_j = s_j = acc_b = s_b = None
    for l in range(NUM_LAYERS):
        lp = p['layers'][l]
        if l == 0:
            xs_b, a_sb, a_db, msb, mdb = _PREP_B0(
                bar_x, p['be_W'], p['be_b'],
                lp['bj']['W'], lp['bj']['att_src'],
                lp['jb']['W'], lp['jb']['att_dst'])
            xs_j, a_sj, a_dj, msj, mdj = _PREP_J0(
                joint_x, p['je_W'], p['je_b'],
                lp['jb']['W'], lp['jb']['att_src'],
                lp['bj']['W'], lp['bj']['att_dst'])
        else:
            lpp = p['layers'][l - 1]
            xs_b, a_sb, a_db, msb, mdb = _PREP_B(
                acc_b, s_b, lpp['jb']['bias'],
                lp['bj']['W'], lp['bj']['att_src'],
                lp['jb']['W'], lp['jb']['att_dst'])
            xs_j, a_sj, a_dj, msj, mdj = _PREP_J(
                acc_j, s_j, lpp['bj']['bias'],
                lp['jb']['W'], lp['jb']['att_src'],
                lp['bj']['W'], lp['bj']['att_dst'])

        acc_j, s_j = _AGG_BJ(srcb_bj, dstb_bj, offs_bj, a_sb[:, 0], a_dj[:, 0],
                             xs_b, _gvec(msb, mdj))
        acc_b, s_b = _AGG_JB(srcb_jb, dstb_jb, offs_jb, a_sj[:, 0], a_db[:, 0],
                             xs_j, _gvec(msj, mdb))
        s_j = s_j[:, None]
        s_b = s_b[:, None]

    probs, value = _HEAD(
        acc_b, s_b, p['layers'][NUM_LAYERS - 1]['jb']['bias'],
        p['a_W1'], p['a_b1'], p['a_g'], p['a_be'], p['a_W2'], p['a_b2'],
        bar_batch[:, None],
        p['c_W1'], p['c_b1'], p['c_g'], p['c_be'], p['c_W2'], p['c_b2'])
    return probs[:, 0], value


# speedup vs baseline: 1.0564x; 1.0564x over previous
import functools

import jax
import jax.numpy as jnp
from jax import lax
from jax.experimental import pallas as pl
from jax.experimental.pallas import tpu as pltpu
from jax.experimental.pallas import tpu_sc as plsc

HIDDEN = 64
NUM_LAYERS = 3
N_JOINT = 50000
N_BAR = 400000
E = 800000
NUM_GRAPHS = 16

NC = 2
NS = 16
NW = NC * NS
CHUNK = 25000
CHUNK_A = 25088
STRIPE = 1568
LAST_STRIPE = CHUNK - 15 * STRIPE
EP = E + 1024
DUMP = EP - 16
BB = 128
PER_TILE = E // NW


def _lane_i32(row, idx, lanes):
    return jnp.max(jnp.where(lanes == idx, row, jnp.int32(-2147483647)))


def _mesh():
    return plsc.VectorSubcoreMesh(core_axis_name="c", subcore_axis_name="s",
                                  num_cores=NC, num_subcores=NS)


def _make_count(nchunk):
    @functools.partial(
        pl.kernel,
        out_type=jax.ShapeDtypeStruct((NW, 16), jnp.int32),
        mesh=_mesh(),
        compiler_params=pltpu.CompilerParams(needs_layout_passes=False),
        scratch_types=[
            pltpu.VMEM((2048,), jnp.int32),
            pltpu.VMEM((16,), jnp.int32),
        ],
    )
    def k(dst_hbm, counts_hbm, dbuf, crow):
        c = lax.axis_index("c")
        s = lax.axis_index("s")
        wid = c * NS + s
        lanes = lax.iota(jnp.int32, 16)
        base0 = wid * PER_TILE

        def groups(ngroups, limit, counts):
            def g_body(g, counts):
                d16 = dbuf[pl.ds(g * 16, 16)]
                valid = (g * 16 + lanes) < limit
                bkt = jnp.where(valid, d16 // CHUNK, 0)

                def k_body(kk, counts):
                    mk = valid & (bkt == kk)
                    pc = plsc.all_reduce_population_count(mk)
                    return counts + jnp.where(lanes == kk, pc, 0)

                return lax.fori_loop(0, nchunk, k_body, counts)

            return lax.fori_loop(0, ngroups, g_body, counts)

        def r_body(r, counts):
            pltpu.sync_copy(dst_hbm.at[pl.ds(base0 + r * 2048, 2048)], dbuf)
            return groups(128, 2048, counts)

        counts = lax.fori_loop(0, 12, r_body, jnp.zeros((16,), jnp.int32))
        pltpu.sync_copy(dst_hbm.at[pl.ds(base0 + 12 * 2048, 424)],
                        dbuf.at[pl.ds(0, 424)])
        counts = groups(27, 424, counts)
        crow[...] = counts
        pltpu.sync_copy(crow, counts_hbm.at[wid])

    return k


def _make_place(nchunk):
    @functools.partial(
        pl.kernel,
        out_type=(
            jax.ShapeDtypeStruct((EP,), jnp.int32),
            jax.ShapeDtypeStruct((EP,), jnp.int32),
            jax.ShapeDtypeStruct((2, 16), jnp.int32),
        ),
        mesh=_mesh(),
        compiler_params=pltpu.CompilerParams(needs_layout_passes=False),
        scratch_types=[
            pltpu.VMEM((NW, 16), jnp.int32),
            pltpu.VMEM((16,), jnp.int32),
            pltpu.VMEM((128,), jnp.int32),
            pltpu.VMEM((128,), jnp.int32),
            pltpu.VMEM((128,), jnp.int32),
            pltpu.VMEM((128,), jnp.int32),
            pltpu.VMEM((128,), jnp.int32),
            pltpu.VMEM((2, 16), jnp.int32),
            pltpu.SemaphoreType.DMA,
        ],
    )
    def k(src_hbm, dst_hbm, counts_hbm, srcb, dstb, offs,
          cts, wb, ssrc, sdst, pos_st, src_st, dst_st, offs_st, sem):
        c = lax.axis_index("c")
        s = lax.axis_index("s")
        wid = c * NS + s
        lanes = lax.iota(jnp.int32, 16)
        pltpu.sync_copy(counts_hbm, cts)

        def tot_body(t, total):
            return total + cts[t, :]

        total = lax.fori_loop(0, NW, tot_body, jnp.zeros((16,), jnp.int32))
        pad8 = ((total + 7) // 8) * 8
        starts = plsc.cumsum(pad8) - pad8

        def pref_body(t, pref):
            tv = jnp.broadcast_to(t, (16,)).astype(jnp.int32)
            wv = jnp.broadcast_to(wid, (16,)).astype(jnp.int32)
            return pref + jnp.where(tv < wv, cts[t, :], 0)

        pref = lax.fori_loop(0, NW, pref_body, jnp.zeros((16,), jnp.int32))
        wb[...] = starts + pref

        @pl.when(wid == 0)
        def _():
            offs_st[0, :] = starts
            offs_st[1, :] = total
            pltpu.sync_copy(offs_st, offs)

        base0 = wid * PER_TILE

        def do_block(limit):
            def g_body(g, _):
                off = g * 16
                s16 = ssrc[pl.ds(off, 16)]
                d16 = sdst[pl.ds(off, 16)]
                valid = (off + lanes) < limit
                bkt = jnp.where(valid, d16 // CHUNK, 0)
                wbg = plsc.load_gather(wb, [bkt])

                def k_body(kk, carry):
                    rank, inc = carry
                    mk = valid & (bkt == kk)
                    csum = plsc.cumsum(mk.astype(jnp.int32))
                    rank = rank + jnp.where(mk, csum - 1, 0)
                    inc = inc + jnp.where(
                        lanes == kk, plsc.all_reduce_population_count(mk), 0)
                    return (rank, inc)

                rank, inc = lax.fori_loop(
                    0, nchunk, k_body,
                    (jnp.zeros((16,), jnp.int32), jnp.zeros((16,), jnp.int32)))
                pos = jnp.where(valid, wbg + rank, DUMP + lanes)
                wb[...] = wb[...] + inc
                pos_st[pl.ds(off, 16)] = pos
                src_st[pl.ds(off, 16)] = s16
                dst_st[pl.ds(off, 16)] = d16
                return 0

            lax.fori_loop(0, 8, g_body, 0)
            pltpu.async_copy(src_st, srcb.at[pos_st], sem).wait()
            pltpu.async_copy(dst_st, dstb.at[pos_st], sem).wait()

        def r_body(r, _):
            base = base0 + r * 128
            pltpu.sync_copy(src_hbm.at[pl.ds(base, 128)], ssrc)
            pltpu.sync_copy(dst_hbm.at[pl.ds(base, 128)], sdst)
            do_block(128)
            return 0

        lax.fori_loop(0, 195, r_body, 0)
        tb = base0 + 195 * 128
        pltpu.sync_copy(src_hbm.at[pl.ds(tb, 40)], ssrc.at[pl.ds(0, 40)])
        pltpu.sync_copy(dst_hbm.at[pl.ds(tb, 40)], sdst.at[pl.ds(0, 40)])
        do_block(40)

    return k


def _make_agg(nsrc, ndst, nchunk):
    chunks_per_sc = nchunk // NC

    @functools.partial(
        pl.kernel,
        out_type=(
            jax.ShapeDtypeStruct((ndst, HIDDEN), jnp.float32),
            jax.ShapeDtypeStruct((ndst,), jnp.float32),
        ),
        mesh=_mesh(),
        compiler_params=pltpu.CompilerParams(needs_layout_passes=False,
                                             use_tc_tiling_on_sc=False),
        scratch_types=[
            pltpu.VMEM_SHARED((CHUNK_A, HIDDEN), jnp.float32),
            pltpu.VMEM_SHARED((CHUNK_A,), jnp.float32),
            pltpu.VMEM((BB,), jnp.float32),
            pltpu.VMEM((BB,), jnp.int32),
            pltpu.VMEM((BB,), jnp.int32),
            pltpu.VMEM((BB,), jnp.int32),
            pltpu.VMEM((BB,), jnp.float32),
            pltpu.VMEM((BB,), jnp.float32),
            pltpu.VMEM((BB, HIDDEN), jnp.float32),
            pltpu.VMEM((STRIPE,), jnp.float32),
            pltpu.VMEM((2, 16), jnp.int32),
            pltpu.VMEM((16,), jnp.float32),
            pltpu.SemaphoreType.DMA,
            pltpu.SemaphoreType.DMA,
        ],
    )
    def k(srcb, dstb, offs, a_s, a_d, xs, gv, acc_hbm, s_hbm,
          out_sp, s_sp, adv_b, src_v, dst_v, dstl_v, asv, w_v, rows,
          szbuf, offs_v, gv_v, sem, sem2):
        c = lax.axis_index("c")
        t = lax.axis_index("s")
        lanes = lax.iota(jnp.int32, 16)
        pltpu.sync_copy(offs, offs_v)
        pltpu.sync_copy(gv, gv_v)
        gvec = gv_v[...]

        def z1(i, _):
            szbuf[pl.ds(i * 16, 16)] = jnp.zeros((16,), jnp.float32)
            return 0

        lax.fori_loop(0, STRIPE // 16, z1, 0)

        def z2(q, _):
            rows[q // 4, pl.ds((q % 4) * 16, 16)] = jnp.zeros((16,), jnp.float32)
            return 0

        r0 = pl.multiple_of(t * STRIPE, 8)
        for cc in range(chunks_per_sc):
            ci = c * chunks_per_sc + cc
            lo = pl.multiple_of(ci * CHUNK, 8)
            start = pl.multiple_of(_lane_i32(offs_v[0, :], ci, lanes), 8)
            cnt = _lane_i32(offs_v[1, :], ci, lanes)
            lax.fori_loop(0, BB * 4, z2, 0)
            for kk in range(12):
                pltpu.sync_copy(rows, out_sp.at[pl.ds(r0 + kk * 128, 128)])
            pltpu.sync_copy(rows.at[pl.ds(0, 32)],
                            out_sp.at[pl.ds(r0 + 12 * 128, 32)])
            pltpu.sync_copy(szbuf, s_sp.at[pl.ds(r0, STRIPE)])
            plsc.subcore_barrier()

            share = pl.multiple_of(((cnt + 15) // 16 + 7) // 8 * 8, 8)
            t0 = start + t * share
            nb = (share + BB - 1) // BB
            mylim = jnp.minimum(cnt - t * share, share)

            def b_body(i, _):
                base = pl.multiple_of(t0 + i * BB, 8)
                lim = mylim - i * BB
                cp1 = pltpu.async_copy(srcb.at[pl.ds(base, BB)], src_v, sem)
                cp2 = pltpu.async_copy(dstb.at[pl.ds(base, BB)], dst_v, sem)
                cp1.wait()
                cp2.wait()

                def g_body(g, _):
                    off = g * 16
                    valid = (off + lanes) < lim
                    d16 = dst_v[pl.ds(off, 16)]
                    s16 = jnp.where(valid, src_v[pl.ds(off, 16)], 0)
                    dg = jnp.where(valid, d16, lo)
                    src_v[pl.ds(off, 16)] = s16
                    dst_v[pl.ds(off, 16)] = dg
                    dstl_v[pl.ds(off, 16)] = dg - lo
                    return 0

                lax.fori_loop(0, BB // 16, g_body, 0)
                cp_as = pltpu.async_copy(a_s.at[src_v], asv, sem)
                cp_ad = pltpu.async_copy(a_d.at[dst_v], adv_b, sem)
                cp_xs = pltpu.async_copy(xs.at[src_v], rows, sem2)
                cp_as.wait()
                cp_ad.wait()

                def g2(g, _):
                    off = g * 16
                    valid = (off + lanes) < lim
                    pre = asv[pl.ds(off, 16)] + adv_b[pl.ds(off, 16)]
                    e = jnp.where(pre >= 0, pre, 0.2 * pre)
                    w = jnp.where(valid, jnp.exp(e - gvec), 0.0)
                    w_v[pl.ds(off, 16)] = w
                    return 0

                lax.fori_loop(0, BB // 16, g2, 0)
                cp_sa = pltpu.async_copy(w_v, s_sp.at[dstl_v], sem, add=True)
                cp_xs.wait()

                def wmul(g, _):
                    off = g * 16
                    w16 = w_v[pl.ds(off, 16)]
                    row_idx = off + lanes

                    def d_body(d, _):
                        col = jnp.broadcast_to(d, (16,)).astype(jnp.int32)
                        vals = plsc.load_gather(rows, [row_idx, col])
                        plsc.store_scatter(rows, [row_idx, col], vals * w16)
                        return 0

                    lax.fori_loop(0, HIDDEN, d_body, 0)
                    return 0

                lax.fori_loop(0, BB // 16, wmul, 0)
                cp_oa = pltpu.async_copy(rows, out_sp.at[dstl_v], sem2, add=True)
                cp_sa.wait()
                cp_oa.wait()
                return 0

            lax.fori_loop(0, nb, b_body, 0)
            plsc.subcore_barrier()

            @pl.when(t < 15)
            def _():
                pltpu.sync_copy(out_sp.at[pl.ds(r0, STRIPE)],
                                acc_hbm.at[pl.ds(lo + r0, STRIPE)])
                pltpu.sync_copy(s_sp.at[pl.ds(r0, STRIPE)],
                                s_hbm.at[pl.ds(lo + r0, STRIPE)])

            @pl.when(t == 15)
            def _():
                pltpu.sync_copy(out_sp.at[pl.ds(r0, LAST_STRIPE)],
                                acc_hbm.at[pl.ds(lo + r0, LAST_STRIPE)])
                pltpu.sync_copy(s_sp.at[pl.ds(r0, LAST_STRIPE)],
                                s_hbm.at[pl.ds(lo + r0, LAST_STRIPE)])

    return k


_COUNT = {n: _make_count(n) for n in (2, 16)}
_PLACE = {n: _make_place(n) for n in (2, 16)}
_AGG_BJ = _make_agg(N_BAR, N_JOINT, 2)
_AGG_JB = _make_agg(N_JOINT, N_BAR, 16)

_NEG = -3.4e38


def _make_prep(n, blk, first, f_raw):
    grid = (n // blk,)

    def body(*refs):
        if first:
            (raw, w_enc, b_enc, w_a, att_a, w_b, att_b,
             xs_o, as_o, ad_o, ms_o, md_o) = refs
            x = raw[...] @ w_enc[...] + b_enc[...]
        else:
            (acc, s, bias, w_a, att_a, w_b, att_b,
             xs_o, as_o, ad_o, ms_o, md_o) = refs
            x = jax.nn.relu(acc[...] / (s[...] + 1e-16) + bias[...])
        xs = x @ w_a[...]
        a_s = jnp.sum(xs * att_a[...], axis=1, keepdims=True)
        v = w_b[...] @ att_b[...][:, None]
        a_d = x @ v
        xs_o[...] = xs
        as_o[...] = a_s
        ad_o[...] = a_d
        i = pl.program_id(0)

        @pl.when(i == 0)
        def _():
            ms_o[...] = jnp.full((1, 1), _NEG, jnp.float32)
            md_o[...] = jnp.full((1, 1), _NEG, jnp.float32)

        ms_o[...] = jnp.maximum(ms_o[...], jnp.max(a_s))
        md_o[...] = jnp.maximum(md_o[...], jnp.max(a_d))

    vec = pl.BlockSpec((HIDDEN,), lambda i: (0,))
    mat = pl.BlockSpec((HIDDEN, HIDDEN), lambda i: (0, 0))
    one = pl.BlockSpec((1, 1), lambda i: (0, 0))
    if first:
        in_specs = [pl.BlockSpec((blk, f_raw), lambda i: (i, 0)),
                    pl.BlockSpec((f_raw, HIDDEN), lambda i: (0, 0)), vec]
    else:
        in_specs = [pl.BlockSpec((blk, HIDDEN), lambda i: (i, 0)),
                    pl.BlockSpec((blk, 1), lambda i: (i, 0)), vec]
    in_specs += [mat, vec, mat, vec]
    return pl.pallas_call(
        body,
        grid=grid,
        in_specs=in_specs,
        out_specs=[
            pl.BlockSpec((blk, HIDDEN), lambda i: (i, 0)),
            pl.BlockSpec((blk, 1), lambda i: (i, 0)),
            pl.BlockSpec((blk, 1), lambda i: (i, 0)),
            one, one,
        ],
        out_shape=[
            jax.ShapeDtypeStruct((n, HIDDEN), jnp.float32),
            jax.ShapeDtypeStruct((n, 1), jnp.float32),
            jax.ShapeDtypeStruct((n, 1), jnp.float32),
            jax.ShapeDtypeStruct((1, 1), jnp.float32),
            jax.ShapeDtypeStruct((1, 1), jnp.float32),
        ],
    )


_PREP_J0 = _make_prep(N_JOINT, 2000, True, 6)
_PREP_B0 = _make_prep(N_BAR, 2000, True, 4)
_PREP_J = _make_prep(N_JOINT, 2000, False, 0)
_PREP_B = _make_prep(N_BAR, 2000, False, 0)


_HBLK = 2000


def _head_body(acc, s, bias, aw1, ab1, ag, abe, aw2, ab2, batch,
               cw1, cb1, cg, cbe, cw2, cb2,
               probs_o, value_o, pooled_sc, cnt_sc):
    i = pl.program_id(0)
    b = jax.nn.relu(acc[...] / (s[...] + 1e-16) + bias[...])
    h = b @ aw1[...] + ab1[...]
    m = jnp.mean(h, axis=-1, keepdims=True)
    v = jnp.mean((h - m) ** 2, axis=-1, keepdims=True)
    h = (h - m) / jnp.sqrt(v + 1e-5) * ag[...] + abe[...]
    h = jax.nn.relu(h)
    probs_o[...] = jax.nn.sigmoid(h @ aw2[...] + ab2[...])

    @pl.when(i == 0)
    def _():
        pooled_sc[...] = jnp.zeros_like(pooled_sc)
        cnt_sc[...] = jnp.zeros_like(cnt_sc)

    gids = lax.broadcasted_iota(jnp.int32, (NUM_GRAPHS, 1), 0)
    onehot = (batch[...][:, 0][None, :] == gids).astype(jnp.float32)
    pooled_sc[...] = pooled_sc[...] + onehot @ b
    cnt_sc[...] = cnt_sc[...] + jnp.sum(onehot, axis=1, keepdims=True)

    @pl.when(i == (N_BAR // _HBLK) - 1)
    def _():
        pooled = pooled_sc[...] / jnp.maximum(cnt_sc[...], 1.0)
        h2 = pooled @ cw1[...] + cb1[...]
        m2 = jnp.mean(h2, axis=-1, keepdims=True)
        v2 = jnp.mean((h2 - m2) ** 2, axis=-1, keepdims=True)
        h2 = (h2 - m2) / jnp.sqrt(v2 + 1e-5) * cg[...] + cbe[...]
        h2 = jax.nn.relu(h2)
        value_o[...] = jnp.tanh(h2 @ cw2[...] + cb2[...])[:, 0]


def _make_head():
    vec = pl.BlockSpec((HIDDEN,), lambda i: (0,))
    mat = pl.BlockSpec((HIDDEN, HIDDEN), lambda i: (0, 0))
    col = pl.BlockSpec((HIDDEN, 1), lambda i: (0, 0))
    one = pl.BlockSpec((1,), lambda i: (0,))
    return pl.pallas_call(
        _head_body,
        grid=(N_BAR // _HBLK,),
        in_specs=[
            pl.BlockSpec((_HBLK, HIDDEN), lambda i: (i, 0)),
            pl.BlockSpec((_HBLK, 1), lambda i: (i, 0)),
            vec,
            mat, vec, vec, vec, col, one,
            pl.BlockSpec((_HBLK, 1), lambda i: (i, 0)),
            mat, vec, vec, vec, col, one,
        ],
        out_specs=[
            pl.BlockSpec((_HBLK, 1), lambda i: (i, 0)),
            pl.BlockSpec((NUM_GRAPHS,), lambda i: (0,)),
        ],
        out_shape=[
            jax.ShapeDtypeStruct((N_BAR, 1), jnp.float32),
            jax.ShapeDtypeStruct((NUM_GRAPHS,), jnp.float32),
        ],
        scratch_shapes=[
            pltpu.VMEM((NUM_GRAPHS, HIDDEN), jnp.float32),
            pltpu.VMEM((NUM_GRAPHS, 1), jnp.float32),
        ],
    )


_HEAD = _make_head()


def _gvec(ms, md):
    g = jax.nn.leaky_relu(ms[0, 0] + md[0, 0], negative_slope=0.2)
    return jnp.full((16,), g, jnp.float32)


def kernel(joint_x, bar_x, params, bj_src, bj_dst, jb_src, jb_dst, bar_batch):
    p = params
    cnt_bj = _COUNT[2](bj_dst)
    srcb_bj, dstb_bj, offs_bj = _PLACE[2](bj_src, bj_dst, cnt_bj)
    cnt_jb = _COUNT[16](jb_dst)
    srcb_jb, dstb_jb, offs_jb = _PLACE[16](jb_src, jb_dst, cnt_jb)

    acc_j = s_j = acc_b = s_b = None
    for l in range(NUM_LAYERS):
        lp = p['layers'][l]
        if l == 0:
            xs_b, a_sb, a_db, msb, mdb = _PREP_B0(
                bar_x, p['be_W'], p['be_b'],
                lp['bj']['W'], lp['bj']['att_src'],
                lp['jb']['W'], lp['jb']['att_dst'])
            xs_j, a_sj, a_dj, msj, mdj = _PREP_J0(
                joint_x, p['je_W'], p['je_b'],
                lp['jb']['W'], lp['jb']['att_src'],
                lp['bj']['W'], lp['bj']['att_dst'])
        else:
            lpp = p['layers'][l - 1]
            xs_b, a_sb, a_db, msb, mdb = _PREP_B(
                acc_b, s_b, lpp['jb']['bias'],
                lp['bj']['W'], lp['bj']['att_src'],
                lp['jb']['W'], lp['jb']['att_dst'])
            xs_j, a_sj, a_dj, msj, mdj = _PREP_J(
                acc_j, s_j, lpp['bj']['bias'],
                lp['jb']['W'], lp['jb']['att_src'],
                lp['bj']['W'], lp['bj']['att_dst'])

        acc_j, s_j = _AGG_BJ(srcb_bj, dstb_bj, offs_bj, a_sb[:, 0], a_dj[:, 0],
                             xs_b, _gvec(msb, mdj))
        acc_b, s_b = _AGG_JB(srcb_jb, dstb_jb, offs_jb, a_sj[:, 0], a_db[:, 0],
                             xs_j, _gvec(msj, mdb))
        s_j = s_j[:, None]
        s_b = s_b[:, None]

    probs, value = _HEAD(
        acc_b, s_b, p['layers'][NUM_LAYERS - 1]['jb']['bias'],
        p['a_W1'], p['a_b1'], p['a_g'], p['a_be'], p['a_W2'], p['a_b2'],
        bar_batch[:, None],
        p['c_W1'], p['c_b1'], p['c_g'], p['c_be'], p['c_W2'], p['c_b2'])
    return probs[:, 0], value


# speedup vs baseline: 1.6823x; 1.5924x over previous
import functools

import jax
import jax.numpy as jnp
from jax import lax
from jax.experimental import pallas as pl
from jax.experimental.pallas import tpu as pltpu
from jax.experimental.pallas import tpu_sc as plsc

HIDDEN = 64
NUM_LAYERS = 3
N_JOINT = 50000
N_BAR = 400000
E = 800000
NUM_GRAPHS = 16

NC = 2
NS = 16
NW = NC * NS
CHUNK = 25000
CHUNK_A = 25088
STRIPE = 1568
LAST_STRIPE = CHUNK - 15 * STRIPE
EP = E + 1024
DUMP = EP - 16
BB = 128
PER_TILE = E // NW


def _lane_i32(row, idx, lanes):
    return jnp.max(jnp.where(lanes == idx, row, jnp.int32(-2147483647)))


def _mesh():
    return plsc.VectorSubcoreMesh(core_axis_name="c", subcore_axis_name="s",
                                  num_cores=NC, num_subcores=NS)


def _make_count(nchunk):
    @functools.partial(
        pl.kernel,
        out_type=jax.ShapeDtypeStruct((NW, 16), jnp.int32),
        mesh=_mesh(),
        compiler_params=pltpu.CompilerParams(needs_layout_passes=False),
        scratch_types=[
            pltpu.VMEM((2048,), jnp.int32),
            pltpu.VMEM((16,), jnp.int32),
        ],
    )
    def k(dst_hbm, counts_hbm, dbuf, crow):
        c = lax.axis_index("c")
        s = lax.axis_index("s")
        wid = c * NS + s
        lanes = lax.iota(jnp.int32, 16)
        base0 = wid * PER_TILE

        def groups(ngroups, limit, counts):
            def g_body(g, counts):
                d16 = dbuf[pl.ds(g * 16, 16)]
                valid = (g * 16 + lanes) < limit
                bkt = jnp.where(valid, d16 // CHUNK, 0)

                def k_body(kk, counts):
                    mk = valid & (bkt == kk)
                    pc = plsc.all_reduce_population_count(mk)
                    return counts + jnp.where(lanes == kk, pc, 0)

                return lax.fori_loop(0, nchunk, k_body, counts)

            return lax.fori_loop(0, ngroups, g_body, counts)

        def r_body(r, counts):
            pltpu.sync_copy(dst_hbm.at[pl.ds(base0 + r * 2048, 2048)], dbuf)
            return groups(128, 2048, counts)

        counts = lax.fori_loop(0, 12, r_body, jnp.zeros((16,), jnp.int32))
        pltpu.sync_copy(dst_hbm.at[pl.ds(base0 + 12 * 2048, 424)],
                        dbuf.at[pl.ds(0, 424)])
        counts = groups(27, 424, counts)
        crow[...] = counts
        pltpu.sync_copy(crow, counts_hbm.at[wid])

    return k


def _make_place(nchunk):
    @functools.partial(
        pl.kernel,
        out_type=(
            jax.ShapeDtypeStruct((EP,), jnp.int32),
            jax.ShapeDtypeStruct((EP,), jnp.int32),
            jax.ShapeDtypeStruct((2, 16), jnp.int32),
        ),
        mesh=_mesh(),
        compiler_params=pltpu.CompilerParams(needs_layout_passes=False),
        scratch_types=[
            pltpu.VMEM((NW, 16), jnp.int32),
            pltpu.VMEM((16,), jnp.int32),
            pltpu.VMEM((128,), jnp.int32),
            pltpu.VMEM((128,), jnp.int32),
            pltpu.VMEM((128,), jnp.int32),
            pltpu.VMEM((128,), jnp.int32),
            pltpu.VMEM((128,), jnp.int32),
            pltpu.VMEM((2, 16), jnp.int32),
            pltpu.SemaphoreType.DMA,
        ],
    )
    def k(src_hbm, dst_hbm, counts_hbm, srcb, dstb, offs,
          cts, wb, ssrc, sdst, pos_st, src_st, dst_st, offs_st, sem):
        c = lax.axis_index("c")
        s = lax.axis_index("s")
        wid = c * NS + s
        lanes = lax.iota(jnp.int32, 16)
        pltpu.sync_copy(counts_hbm, cts)

        def tot_body(t, total):
            return total + cts[t, :]

        total = lax.fori_loop(0, NW, tot_body, jnp.zeros((16,), jnp.int32))
        pad8 = ((total + 7) // 8) * 8
        starts = plsc.cumsum(pad8) - pad8

        def pref_body(t, pref):
            tv = jnp.broadcast_to(t, (16,)).astype(jnp.int32)
            wv = jnp.broadcast_to(wid, (16,)).astype(jnp.int32)
            return pref + jnp.where(tv < wv, cts[t, :], 0)

        pref = lax.fori_loop(0, NW, pref_body, jnp.zeros((16,), jnp.int32))
        wb[...] = starts + pref

        @pl.when(wid == 0)
        def _():
            offs_st[0, :] = starts
            offs_st[1, :] = total
            pltpu.sync_copy(offs_st, offs)

        base0 = wid * PER_TILE

        def do_block(limit):
            def g_body(g, _):
                off = g * 16
                s16 = ssrc[pl.ds(off, 16)]
                d16 = sdst[pl.ds(off, 16)]
                valid = (off + lanes) < limit
                bkt = jnp.where(valid, d16 // CHUNK, 0)
                wbg = plsc.load_gather(wb, [bkt])

                def k_body(kk, carry):
                    rank, inc = carry
                    mk = valid & (bkt == kk)
                    csum = plsc.cumsum(mk.astype(jnp.int32))
                    rank = rank + jnp.where(mk, csum - 1, 0)
                    inc = inc + jnp.where(
                        lanes == kk, plsc.all_reduce_population_count(mk), 0)
                    return (rank, inc)

                rank, inc = lax.fori_loop(
                    0, nchunk, k_body,
                    (jnp.zeros((16,), jnp.int32), jnp.zeros((16,), jnp.int32)))
                pos = jnp.where(valid, wbg + rank, DUMP + lanes)
                wb[...] = wb[...] + inc
                pos_st[pl.ds(off, 16)] = pos
                src_st[pl.ds(off, 16)] = s16
                dst_st[pl.ds(off, 16)] = d16
                return 0

            lax.fori_loop(0, 8, g_body, 0)
            pltpu.async_copy(src_st, srcb.at[pos_st], sem).wait()
            pltpu.async_copy(dst_st, dstb.at[pos_st], sem).wait()

        def r_body(r, _):
            base = base0 + r * 128
            pltpu.sync_copy(src_hbm.at[pl.ds(base, 128)], ssrc)
            pltpu.sync_copy(dst_hbm.at[pl.ds(base, 128)], sdst)
            do_block(128)
            return 0

        lax.fori_loop(0, 195, r_body, 0)
        tb = base0 + 195 * 128
        pltpu.sync_copy(src_hbm.at[pl.ds(tb, 40)], ssrc.at[pl.ds(0, 40)])
        pltpu.sync_copy(dst_hbm.at[pl.ds(tb, 40)], sdst.at[pl.ds(0, 40)])
        do_block(40)

    return k


def _make_agg(nsrc, ndst, nchunk):
    chunks_per_sc = nchunk // NC

    @functools.partial(
        pl.kernel,
        out_type=(
            jax.ShapeDtypeStruct((ndst, HIDDEN), jnp.float32),
            jax.ShapeDtypeStruct((ndst,), jnp.float32),
        ),
        mesh=_mesh(),
        compiler_params=pltpu.CompilerParams(needs_layout_passes=False,
                                             use_tc_tiling_on_sc=False),
        scratch_types=[
            pltpu.VMEM_SHARED((CHUNK_A, HIDDEN), jnp.float32),
            pltpu.VMEM_SHARED((CHUNK_A,), jnp.float32),
            pltpu.VMEM((BB,), jnp.float32),
            pltpu.VMEM((BB,), jnp.int32),
            pltpu.VMEM((BB,), jnp.int32),
            pltpu.VMEM((BB,), jnp.int32),
            pltpu.VMEM((BB,), jnp.float32),
            pltpu.VMEM((BB,), jnp.float32),
            pltpu.VMEM((BB, HIDDEN), jnp.float32),
            pltpu.VMEM((STRIPE,), jnp.float32),
            pltpu.VMEM((2, 16), jnp.int32),
            pltpu.VMEM((16,), jnp.float32),
            pltpu.SemaphoreType.DMA,
            pltpu.SemaphoreType.DMA,
        ],
    )
    def k(srcb, dstb, offs, a_s, a_d, xs, gv, acc_hbm, s_hbm,
          out_sp, s_sp, adv_b, src_v, dst_v, dstl_v, asv, w_v, rows,
          szbuf, offs_v, gv_v, sem, sem2):
        c = lax.axis_index("c")
        t = lax.axis_index("s")
        lanes = lax.iota(jnp.int32, 16)
        pltpu.sync_copy(offs, offs_v)
        pltpu.sync_copy(gv, gv_v)
        gvec = gv_v[...]

        def z1(i, _):
            szbuf[pl.ds(i * 16, 16)] = jnp.zeros((16,), jnp.float32)
            return 0

        lax.fori_loop(0, STRIPE // 16, z1, 0)

        def z2(q, _):
            rows[q // 4, pl.ds((q % 4) * 16, 16)] = jnp.zeros((16,), jnp.float32)
            return 0

        r0 = pl.multiple_of(t * STRIPE, 8)
        for cc in range(chunks_per_sc):
            ci = c * chunks_per_sc + cc
            lo = pl.multiple_of(ci * CHUNK, 8)
            start = pl.multiple_of(_lane_i32(offs_v[0, :], ci, lanes), 8)
            cnt = _lane_i32(offs_v[1, :], ci, lanes)
            lax.fori_loop(0, BB * 4, z2, 0)
            for kk in range(12):
                pltpu.sync_copy(rows, out_sp.at[pl.ds(r0 + kk * 128, 128)])
            pltpu.sync_copy(rows.at[pl.ds(0, 32)],
                            out_sp.at[pl.ds(r0 + 12 * 128, 32)])
            pltpu.sync_copy(szbuf, s_sp.at[pl.ds(r0, STRIPE)])
            plsc.subcore_barrier()

            share = pl.multiple_of(((cnt + 15) // 16 + 7) // 8 * 8, 8)
            t0 = start + t * share
            nb = (share + BB - 1) // BB
            mylim = jnp.minimum(cnt - t * share, share)

            def b_body(i, _):
                base = pl.multiple_of(t0 + i * BB, 8)
                lim = mylim - i * BB
                cp1 = pltpu.async_copy(srcb.at[pl.ds(base, BB)], src_v, sem)
                cp2 = pltpu.async_copy(dstb.at[pl.ds(base, BB)], dst_v, sem)
                cp1.wait()
                cp2.wait()

                def g_body(g, _):
                    off = g * 16
                    valid = (off + lanes) < lim
                    d16 = dst_v[pl.ds(off, 16)]
                    s16 = jnp.where(valid, src_v[pl.ds(off, 16)], 0)
                    dg = jnp.where(valid, d16, lo)
                    src_v[pl.ds(off, 16)] = s16
                    dst_v[pl.ds(off, 16)] = dg
                    dstl_v[pl.ds(off, 16)] = dg - lo
                    return 0

                lax.fori_loop(0, BB // 16, g_body, 0)
                cp_as = pltpu.async_copy(a_s.at[src_v], asv, sem)
                cp_ad = pltpu.async_copy(a_d.at[dst_v], adv_b, sem)
                cp_xs = pltpu.async_copy(xs.at[src_v], rows, sem2)
                cp_as.wait()
                cp_ad.wait()

                def g2(g, _):
                    off = g * 16
                    valid = (off + lanes) < lim
                    pre = asv[pl.ds(off, 16)] + adv_b[pl.ds(off, 16)]
                    e = jnp.where(pre >= 0, pre, 0.2 * pre)
                    w = jnp.where(valid, jnp.exp(e - gvec), 0.0)
                    w_v[pl.ds(off, 16)] = w
                    return 0

                lax.fori_loop(0, BB // 16, g2, 0)
                cp_sa = pltpu.async_copy(w_v, s_sp.at[dstl_v], sem, add=True)
                cp_xs.wait()

                def wmul(g, _):
                    off = g * 16
                    w16 = w_v[pl.ds(off, 16)]
                    for e in range(16):
                        ws = w16[e]
                        for kk in range(4):
                            sl = pl.ds(kk * 16, 16)
                            rows[off + e, sl] = rows[off + e, sl] * ws
                    return 0

                lax.fori_loop(0, BB // 16, wmul, 0)
                cp_oa = pltpu.async_copy(rows, out_sp.at[dstl_v], sem2, add=True)
                cp_sa.wait()
                cp_oa.wait()
                return 0

            lax.fori_loop(0, nb, b_body, 0)
            plsc.subcore_barrier()

            @pl.when(t < 15)
            def _():
                pltpu.sync_copy(out_sp.at[pl.ds(r0, STRIPE)],
                                acc_hbm.at[pl.ds(lo + r0, STRIPE)])
                pltpu.sync_copy(s_sp.at[pl.ds(r0, STRIPE)],
                                s_hbm.at[pl.ds(lo + r0, STRIPE)])

            @pl.when(t == 15)
            def _():
                pltpu.sync_copy(out_sp.at[pl.ds(r0, LAST_STRIPE)],
                                acc_hbm.at[pl.ds(lo + r0, LAST_STRIPE)])
                pltpu.sync_copy(s_sp.at[pl.ds(r0, LAST_STRIPE)],
                                s_hbm.at[pl.ds(lo + r0, LAST_STRIPE)])

    return k


_COUNT = {n: _make_count(n) for n in (2, 16)}
_PLACE = {n: _make_place(n) for n in (2, 16)}
_AGG_BJ = _make_agg(N_BAR, N_JOINT, 2)
_AGG_JB = _make_agg(N_JOINT, N_BAR, 16)

_NEG = -3.4e38


def _make_prep(n, blk, first, f_raw):
    grid = (n // blk,)

    def body(*refs):
        if first:
            (raw, w_enc, b_enc, w_a, att_a, w_b, att_b,
             xs_o, as_o, ad_o, ms_o, md_o) = refs
            x = raw[...] @ w_enc[...] + b_enc[...]
        else:
            (acc, s, bias, w_a, att_a, w_b, att_b,
             xs_o, as_o, ad_o, ms_o, md_o) = refs
            x = jax.nn.relu(acc[...] / (s[...] + 1e-16) + bias[...])
        xs = x @ w_a[...]
        a_s = jnp.sum(xs * att_a[...], axis=1, keepdims=True)
        v = w_b[...] @ att_b[...][:, None]
        a_d = x @ v
        xs_o[...] = xs
        as_o[...] = a_s
        ad_o[...] = a_d
        i = pl.program_id(0)

        @pl.when(i == 0)
        def _():
            ms_o[...] = jnp.full((1, 1), _NEG, jnp.float32)
            md_o[...] = jnp.full((1, 1), _NEG, jnp.float32)

        ms_o[...] = jnp.maximum(ms_o[...], jnp.max(a_s))
        md_o[...] = jnp.maximum(md_o[...], jnp.max(a_d))

    vec = pl.BlockSpec((HIDDEN,), lambda i: (0,))
    mat = pl.BlockSpec((HIDDEN, HIDDEN), lambda i: (0, 0))
    one = pl.BlockSpec((1, 1), lambda i: (0, 0))
    if first:
        in_specs = [pl.BlockSpec((blk, f_raw), lambda i: (i, 0)),
                    pl.BlockSpec((f_raw, HIDDEN), lambda i: (0, 0)), vec]
    else:
        in_specs = [pl.BlockSpec((blk, HIDDEN), lambda i: (i, 0)),
                    pl.BlockSpec((blk, 1), lambda i: (i, 0)), vec]
    in_specs += [mat, vec, mat, vec]
    return pl.pallas_call(
        body,
        grid=grid,
        in_specs=in_specs,
        out_specs=[
            pl.BlockSpec((blk, HIDDEN), lambda i: (i, 0)),
            pl.BlockSpec((blk, 1), lambda i: (i, 0)),
            pl.BlockSpec((blk, 1), lambda i: (i, 0)),
            one, one,
        ],
        out_shape=[
            jax.ShapeDtypeStruct((n, HIDDEN), jnp.float32),
            jax.ShapeDtypeStruct((n, 1), jnp.float32),
            jax.ShapeDtypeStruct((n, 1), jnp.float32),
            jax.ShapeDtypeStruct((1, 1), jnp.float32),
            jax.ShapeDtypeStruct((1, 1), jnp.float32),
        ],
    )


_PREP_J0 = _make_prep(N_JOINT, 2000, True, 6)
_PREP_B0 = _make_prep(N_BAR, 2000, True, 4)
_PREP_J = _make_prep(N_JOINT, 2000, False, 0)
_PREP_B = _make_prep(N_BAR, 2000, False, 0)


_HBLK = 2000


def _head_body(acc, s, bias, aw1, ab1, ag, abe, aw2, ab2, batch,
               cw1, cb1, cg, cbe, cw2, cb2,
               probs_o, value_o, pooled_sc, cnt_sc):
    i = pl.program_id(0)
    b = jax.nn.relu(acc[...] / (s[...] + 1e-16) + bias[...])
    h = b @ aw1[...] + ab1[...]
    m = jnp.mean(h, axis=-1, keepdims=True)
    v = jnp.mean((h - m) ** 2, axis=-1, keepdims=True)
    h = (h - m) / jnp.sqrt(v + 1e-5) * ag[...] + abe[...]
    h = jax.nn.relu(h)
    probs_o[...] = jax.nn.sigmoid(h @ aw2[...] + ab2[...])

    @pl.when(i == 0)
    def _():
        pooled_sc[...] = jnp.zeros_like(pooled_sc)
        cnt_sc[...] = jnp.zeros_like(cnt_sc)

    gids = lax.broadcasted_iota(jnp.int32, (NUM_GRAPHS, 1), 0)
    onehot = (batch[...][:, 0][None, :] == gids).astype(jnp.float32)
    pooled_sc[...] = pooled_sc[...] + onehot @ b
    cnt_sc[...] = cnt_sc[...] + jnp.sum(onehot, axis=1, keepdims=True)

    @pl.when(i == (N_BAR // _HBLK) - 1)
    def _():
        pooled = pooled_sc[...] / jnp.maximum(cnt_sc[...], 1.0)
        h2 = pooled @ cw1[...] + cb1[...]
        m2 = jnp.mean(h2, axis=-1, keepdims=True)
        v2 = jnp.mean((h2 - m2) ** 2, axis=-1, keepdims=True)
        h2 = (h2 - m2) / jnp.sqrt(v2 + 1e-5) * cg[...] + cbe[...]
        h2 = jax.nn.relu(h2)
        value_o[...] = jnp.tanh(h2 @ cw2[...] + cb2[...])[:, 0]


def _make_head():
    vec = pl.BlockSpec((HIDDEN,), lambda i: (0,))
    mat = pl.BlockSpec((HIDDEN, HIDDEN), lambda i: (0, 0))
    col = pl.BlockSpec((HIDDEN, 1), lambda i: (0, 0))
    one = pl.BlockSpec((1,), lambda i: (0,))
    return pl.pallas_call(
        _head_body,
        grid=(N_BAR // _HBLK,),
        in_specs=[
            pl.BlockSpec((_HBLK, HIDDEN), lambda i: (i, 0)),
            pl.BlockSpec((_HBLK, 1), lambda i: (i, 0)),
            vec,
            mat, vec, vec, vec, col, one,
            pl.BlockSpec((_HBLK, 1), lambda i: (i, 0)),
            mat, vec, vec, vec, col, one,
        ],
        out_specs=[
            pl.BlockSpec((_HBLK, 1), lambda i: (i, 0)),
            pl.BlockSpec((NUM_GRAPHS,), lambda i: (0,)),
        ],
        out_shape=[
            jax.ShapeDtypeStruct((N_BAR, 1), jnp.float32),
            jax.ShapeDtypeStruct((NUM_GRAPHS,), jnp.float32),
        ],
        scratch_shapes=[
            pltpu.VMEM((NUM_GRAPHS, HIDDEN), jnp.float32),
            pltpu.VMEM((NUM_GRAPHS, 1), jnp.float32),
        ],
    )


_HEAD = _make_head()


def _gvec(ms, md):
    g = jax.nn.leaky_relu(ms[0, 0] + md[0, 0], negative_slope=0.2)
    return jnp.full((16,), g, jnp.float32)


def kernel(joint_x, bar_x, params, bj_src, bj_dst, jb_src, jb_dst, bar_batch):
    p = params
    cnt_bj = _COUNT[2](bj_dst)
    srcb_bj, dstb_bj, offs_bj = _PLACE[2](bj_src, bj_dst, cnt_bj)
    cnt_jb = _COUNT[16](jb_dst)
    srcb_jb, dstb_jb, offs_jb = _PLACE[16](jb_src, jb_dst, cnt_jb)

    acc_j = s_j = acc_b = s_b = None
    for l in range(NUM_LAYERS):
        lp = p['layers'][l]
        if l == 0:
            xs_b, a_sb, a_db, msb, mdb = _PREP_B0(
                bar_x, p['be_W'], p['be_b'],
                lp['bj']['W'], lp['bj']['att_src'],
                lp['jb']['W'], lp['jb']['att_dst'])
            xs_j, a_sj, a_dj, msj, mdj = _PREP_J0(
                joint_x, p['je_W'], p['je_b'],
                lp['jb']['W'], lp['jb']['att_src'],
                lp['bj']['W'], lp['bj']['att_dst'])
        else:
            lpp = p['layers'][l - 1]
            xs_b, a_sb, a_db, msb, mdb = _PREP_B(
                acc_b, s_b, lpp['jb']['bias'],
                lp['bj']['W'], lp['bj']['att_src'],
                lp['jb']['W'], lp['jb']['att_dst'])
            xs_j, a_sj, a_dj, msj, mdj = _PREP_J(
                acc_j, s_j, lpp['bj']['bias'],
                lp['jb']['W'], lp['jb']['att_src'],
                lp['bj']['W'], lp['bj']['att_dst'])

        acc_j, s_j = _AGG_BJ(srcb_bj, dstb_bj, offs_bj, a_sb[:, 0], a_dj[:, 0],
                             xs_b, _gvec(msb, mdj))
        acc_b, s_b = _AGG_JB(srcb_jb, dstb_jb, offs_jb, a_sj[:, 0], a_db[:, 0],
                             xs_j, _gvec(msj, mdb))
        s_j = s_j[:, None]
        s_b = s_b[:, None]

    probs, value = _HEAD(
        acc_b, s_b, p['layers'][NUM_LAYERS - 1]['jb']['bias'],
        p['a_W1'], p['a_b1'], p['a_g'], p['a_be'], p['a_W2'], p['a_b2'],
        bar_batch[:, None],
        p['c_W1'], p['c_b1'], p['c_g'], p['c_be'], p['c_W2'], p['c_b2'])
    return probs[:, 0], value


# speedup vs baseline: 1.6968x; 1.0086x over previous
import functools

import jax
import jax.numpy as jnp
from jax import lax
from jax.experimental import pallas as pl
from jax.experimental.pallas import tpu as pltpu
from jax.experimental.pallas import tpu_sc as plsc

HIDDEN = 64
NUM_LAYERS = 3
N_JOINT = 50000
N_BAR = 400000
E = 800000
NUM_GRAPHS = 16

NC = 2
NS = 16
NW = NC * NS
CHUNK = 25000
CHUNK_A = 25088
STRIPE = 1568
LAST_STRIPE = CHUNK - 15 * STRIPE
EP = E + 1024
DUMP = EP - 16
BB = 128
PER_TILE = E // NW


def _lane_i32(row, idx, lanes):
    return jnp.max(jnp.where(lanes == idx, row, jnp.int32(-2147483647)))


def _mesh():
    return plsc.VectorSubcoreMesh(core_axis_name="c", subcore_axis_name="s",
                                  num_cores=NC, num_subcores=NS)


def _make_count(nchunk):
    @functools.partial(
        pl.kernel,
        out_type=jax.ShapeDtypeStruct((NW, 16), jnp.int32),
        mesh=_mesh(),
        compiler_params=pltpu.CompilerParams(needs_layout_passes=False),
        scratch_types=[
            pltpu.VMEM((2048,), jnp.int32),
            pltpu.VMEM((16,), jnp.int32),
        ],
    )
    def k(dst_hbm, counts_hbm, dbuf, crow):
        c = lax.axis_index("c")
        s = lax.axis_index("s")
        wid = c * NS + s
        lanes = lax.iota(jnp.int32, 16)
        base0 = wid * PER_TILE

        def groups(ngroups, limit, counts):
            def g_body(g, counts):
                d16 = dbuf[pl.ds(g * 16, 16)]
                valid = (g * 16 + lanes) < limit
                bkt = jnp.where(valid, d16 // CHUNK, 0)

                for kk in range(nchunk):
                    mk = valid & (bkt == kk)
                    pc = plsc.all_reduce_population_count(mk)
                    counts = counts + jnp.where(lanes == kk, pc, 0)
                return counts

            return lax.fori_loop(0, ngroups, g_body, counts)

        def r_body(r, counts):
            pltpu.sync_copy(dst_hbm.at[pl.ds(base0 + r * 2048, 2048)], dbuf)
            return groups(128, 2048, counts)

        counts = lax.fori_loop(0, 12, r_body, jnp.zeros((16,), jnp.int32))
        pltpu.sync_copy(dst_hbm.at[pl.ds(base0 + 12 * 2048, 424)],
                        dbuf.at[pl.ds(0, 424)])
        counts = groups(27, 424, counts)
        crow[...] = counts
        pltpu.sync_copy(crow, counts_hbm.at[wid])

    return k


def _make_place(nchunk):
    @functools.partial(
        pl.kernel,
        out_type=(
            jax.ShapeDtypeStruct((EP,), jnp.int32),
            jax.ShapeDtypeStruct((EP,), jnp.int32),
            jax.ShapeDtypeStruct((2, 16), jnp.int32),
        ),
        mesh=_mesh(),
        compiler_params=pltpu.CompilerParams(needs_layout_passes=False),
        scratch_types=[
            pltpu.VMEM((NW, 16), jnp.int32),
            pltpu.VMEM((16,), jnp.int32),
            pltpu.VMEM((128,), jnp.int32),
            pltpu.VMEM((128,), jnp.int32),
            pltpu.VMEM((128,), jnp.int32),
            pltpu.VMEM((128,), jnp.int32),
            pltpu.VMEM((128,), jnp.int32),
            pltpu.VMEM((2, 16), jnp.int32),
            pltpu.SemaphoreType.DMA,
        ],
    )
    def k(src_hbm, dst_hbm, counts_hbm, srcb, dstb, offs,
          cts, wb, ssrc, sdst, pos_st, src_st, dst_st, offs_st, sem):
        c = lax.axis_index("c")
        s = lax.axis_index("s")
        wid = c * NS + s
        lanes = lax.iota(jnp.int32, 16)
        pltpu.sync_copy(counts_hbm, cts)

        def tot_body(t, total):
            return total + cts[t, :]

        total = lax.fori_loop(0, NW, tot_body, jnp.zeros((16,), jnp.int32))
        pad8 = ((total + 7) // 8) * 8
        starts = plsc.cumsum(pad8) - pad8

        def pref_body(t, pref):
            tv = jnp.broadcast_to(t, (16,)).astype(jnp.int32)
            wv = jnp.broadcast_to(wid, (16,)).astype(jnp.int32)
            return pref + jnp.where(tv < wv, cts[t, :], 0)

        pref = lax.fori_loop(0, NW, pref_body, jnp.zeros((16,), jnp.int32))
        wb[...] = starts + pref

        @pl.when(wid == 0)
        def _():
            offs_st[0, :] = starts
            offs_st[1, :] = total
            pltpu.sync_copy(offs_st, offs)

        base0 = wid * PER_TILE

        def do_block(limit):
            def g_body(g, _):
                off = g * 16
                s16 = ssrc[pl.ds(off, 16)]
                d16 = sdst[pl.ds(off, 16)]
                valid = (off + lanes) < limit
                bkt = jnp.where(valid, d16 // CHUNK, 0)
                wbg = plsc.load_gather(wb, [bkt])

                rank = jnp.zeros((16,), jnp.int32)
                inc = jnp.zeros((16,), jnp.int32)
                for kk in range(nchunk):
                    mk = valid & (bkt == kk)
                    csum = plsc.cumsum(mk.astype(jnp.int32))
                    rank = rank + jnp.where(mk, csum - 1, 0)
                    inc = inc + jnp.where(
                        lanes == kk, plsc.all_reduce_population_count(mk), 0)
                pos = jnp.where(valid, wbg + rank, DUMP + lanes)
                wb[...] = wb[...] + inc
                pos_st[pl.ds(off, 16)] = pos
                src_st[pl.ds(off, 16)] = s16
                dst_st[pl.ds(off, 16)] = d16
                return 0

            lax.fori_loop(0, 8, g_body, 0)
            pltpu.async_copy(src_st, srcb.at[pos_st], sem).wait()
            pltpu.async_copy(dst_st, dstb.at[pos_st], sem).wait()

        def r_body(r, _):
            base = base0 + r * 128
            pltpu.sync_copy(src_hbm.at[pl.ds(base, 128)], ssrc)
            pltpu.sync_copy(dst_hbm.at[pl.ds(base, 128)], sdst)
            do_block(128)
            return 0

        lax.fori_loop(0, 195, r_body, 0)
        tb = base0 + 195 * 128
        pltpu.sync_copy(src_hbm.at[pl.ds(tb, 40)], ssrc.at[pl.ds(0, 40)])
        pltpu.sync_copy(dst_hbm.at[pl.ds(tb, 40)], sdst.at[pl.ds(0, 40)])
        do_block(40)

    return k


def _make_agg(nsrc, ndst, nchunk):
    chunks_per_sc = nchunk // NC

    @functools.partial(
        pl.kernel,
        out_type=(
            jax.ShapeDtypeStruct((ndst, HIDDEN), jnp.float32),
            jax.ShapeDtypeStruct((ndst,), jnp.float32),
        ),
        mesh=_mesh(),
        compiler_params=pltpu.CompilerParams(needs_layout_passes=False,
                                             use_tc_tiling_on_sc=False),
        scratch_types=[
            pltpu.VMEM_SHARED((CHUNK_A, HIDDEN), jnp.float32),
            pltpu.VMEM_SHARED((CHUNK_A,), jnp.float32),
            pltpu.VMEM((BB,), jnp.float32),
            pltpu.VMEM((BB,), jnp.int32),
            pltpu.VMEM((BB,), jnp.int32),
            pltpu.VMEM((BB,), jnp.int32),
            pltpu.VMEM((BB,), jnp.float32),
            pltpu.VMEM((BB,), jnp.float32),
            pltpu.VMEM((BB, HIDDEN), jnp.float32),
            pltpu.VMEM((STRIPE,), jnp.float32),
            pltpu.VMEM((2, 16), jnp.int32),
            pltpu.VMEM((16,), jnp.float32),
            pltpu.SemaphoreType.DMA,
            pltpu.SemaphoreType.DMA,
        ],
    )
    def k(srcb, dstb, offs, a_s, a_d, xs, gv, acc_hbm, s_hbm,
          out_sp, s_sp, adv_b, src_v, dst_v, dstl_v, asv, w_v, rows,
          szbuf, offs_v, gv_v, sem, sem2):
        c = lax.axis_index("c")
        t = lax.axis_index("s")
        lanes = lax.iota(jnp.int32, 16)
        pltpu.sync_copy(offs, offs_v)
        pltpu.sync_copy(gv, gv_v)
        gvec = gv_v[...]

        def z1(i, _):
            szbuf[pl.ds(i * 16, 16)] = jnp.zeros((16,), jnp.float32)
            return 0

        lax.fori_loop(0, STRIPE // 16, z1, 0)

        def z2(q, _):
            rows[q // 4, pl.ds((q % 4) * 16, 16)] = jnp.zeros((16,), jnp.float32)
            return 0

        r0 = pl.multiple_of(t * STRIPE, 8)
        for cc in range(chunks_per_sc):
            ci = c * chunks_per_sc + cc
            lo = pl.multiple_of(ci * CHUNK, 8)
            start = pl.multiple_of(_lane_i32(offs_v[0, :], ci, lanes), 8)
            cnt = _lane_i32(offs_v[1, :], ci, lanes)
            lax.fori_loop(0, BB * 4, z2, 0)
            for kk in range(12):
                pltpu.sync_copy(rows, out_sp.at[pl.ds(r0 + kk * 128, 128)])
            pltpu.sync_copy(rows.at[pl.ds(0, 32)],
                            out_sp.at[pl.ds(r0 + 12 * 128, 32)])
            pltpu.sync_copy(szbuf, s_sp.at[pl.ds(r0, STRIPE)])
            plsc.subcore_barrier()

            share = pl.multiple_of(((cnt + 15) // 16 + 7) // 8 * 8, 8)
            t0 = start + t * share
            nb = (share + BB - 1) // BB
            mylim = jnp.minimum(cnt - t * share, share)

            def b_body(i, _):
                base = pl.multiple_of(t0 + i * BB, 8)
                lim = mylim - i * BB
                cp1 = pltpu.async_copy(srcb.at[pl.ds(base, BB)], src_v, sem)
                cp2 = pltpu.async_copy(dstb.at[pl.ds(base, BB)], dst_v, sem)
                cp1.wait()
                cp2.wait()

                def g_body(g, _):
                    off = g * 16
                    valid = (off + lanes) < lim
                    d16 = dst_v[pl.ds(off, 16)]
                    s16 = jnp.where(valid, src_v[pl.ds(off, 16)], 0)
                    dg = jnp.where(valid, d16, lo)
                    src_v[pl.ds(off, 16)] = s16
                    dst_v[pl.ds(off, 16)] = dg
                    dstl_v[pl.ds(off, 16)] = dg - lo
                    return 0

                lax.fori_loop(0, BB // 16, g_body, 0)
                cp_as = pltpu.async_copy(a_s.at[src_v], asv, sem)
                cp_ad = pltpu.async_copy(a_d.at[dst_v], adv_b, sem)
                cp_xs = pltpu.async_copy(xs.at[src_v], rows, sem2)
                cp_as.wait()
                cp_ad.wait()

                def g2(g, _):
                    off = g * 16
                    valid = (off + lanes) < lim
                    pre = asv[pl.ds(off, 16)] + adv_b[pl.ds(off, 16)]
                    e = jnp.where(pre >= 0, pre, 0.2 * pre)
                    w = jnp.where(valid, jnp.exp(e - gvec), 0.0)
                    w_v[pl.ds(off, 16)] = w
                    return 0

                lax.fori_loop(0, BB // 16, g2, 0)
                cp_sa = pltpu.async_copy(w_v, s_sp.at[dstl_v], sem, add=True)
                cp_xs.wait()

                def wmul(g, _):
                    off = g * 16
                    w16 = w_v[pl.ds(off, 16)]
                    for e in range(16):
                        ws = w16[e]
                        for kk in range(4):
                            sl = pl.ds(kk * 16, 16)
                            rows[off + e, sl] = rows[off + e, sl] * ws
                    return 0

                lax.fori_loop(0, BB // 16, wmul, 0)
                cp_oa = pltpu.async_copy(rows, out_sp.at[dstl_v], sem2, add=True)
                cp_sa.wait()
                cp_oa.wait()
                return 0

            lax.fori_loop(0, nb, b_body, 0)
            plsc.subcore_barrier()

            @pl.when(t < 15)
            def _():
                pltpu.sync_copy(out_sp.at[pl.ds(r0, STRIPE)],
                                acc_hbm.at[pl.ds(lo + r0, STRIPE)])
                pltpu.sync_copy(s_sp.at[pl.ds(r0, STRIPE)],
                                s_hbm.at[pl.ds(lo + r0, STRIPE)])

            @pl.when(t == 15)
            def _():
                pltpu.sync_copy(out_sp.at[pl.ds(r0, LAST_STRIPE)],
                                acc_hbm.at[pl.ds(lo + r0, LAST_STRIPE)])
                pltpu.sync_copy(s_sp.at[pl.ds(r0, LAST_STRIPE)],
                                s_hbm.at[pl.ds(lo + r0, LAST_STRIPE)])

    return k


_COUNT = {n: _make_count(n) for n in (2, 16)}
_PLACE = {n: _make_place(n) for n in (2, 16)}
_AGG_BJ = _make_agg(N_BAR, N_JOINT, 2)
_AGG_JB = _make_agg(N_JOINT, N_BAR, 16)

_NEG = -3.4e38


def _make_prep(n, blk, first, f_raw):
    grid = (n // blk,)

    def body(*refs):
        if first:
            (raw, w_enc, b_enc, w_a, att_a, w_b, att_b,
             xs_o, as_o, ad_o, ms_o, md_o) = refs
            x = raw[...] @ w_enc[...] + b_enc[...]
        else:
            (acc, s, bias, w_a, att_a, w_b, att_b,
             xs_o, as_o, ad_o, ms_o, md_o) = refs
            x = jax.nn.relu(acc[...] / (s[...] + 1e-16) + bias[...])
        xs = x @ w_a[...]
        a_s = jnp.sum(xs * att_a[...], axis=1, keepdims=True)
        v = w_b[...] @ att_b[...][:, None]
        a_d = x @ v
        xs_o[...] = xs
        as_o[...] = a_s
        ad_o[...] = a_d
        i = pl.program_id(0)

        @pl.when(i == 0)
        def _():
            ms_o[...] = jnp.full((1, 1), _NEG, jnp.float32)
            md_o[...] = jnp.full((1, 1), _NEG, jnp.float32)

        ms_o[...] = jnp.maximum(ms_o[...], jnp.max(a_s))
        md_o[...] = jnp.maximum(md_o[...], jnp.max(a_d))

    vec = pl.BlockSpec((HIDDEN,), lambda i: (0,))
    mat = pl.BlockSpec((HIDDEN, HIDDEN), lambda i: (0, 0))
    one = pl.BlockSpec((1, 1), lambda i: (0, 0))
    if first:
        in_specs = [pl.BlockSpec((blk, f_raw), lambda i: (i, 0)),
                    pl.BlockSpec((f_raw, HIDDEN), lambda i: (0, 0)), vec]
    else:
        in_specs = [pl.BlockSpec((blk, HIDDEN), lambda i: (i, 0)),
                    pl.BlockSpec((blk, 1), lambda i: (i, 0)), vec]
    in_specs += [mat, vec, mat, vec]
    return pl.pallas_call(
        body,
        grid=grid,
        in_specs=in_specs,
        out_specs=[
            pl.BlockSpec((blk, HIDDEN), lambda i: (i, 0)),
            pl.BlockSpec((blk, 1), lambda i: (i, 0)),
            pl.BlockSpec((blk, 1), lambda i: (i, 0)),
            one, one,
        ],
        out_shape=[
            jax.ShapeDtypeStruct((n, HIDDEN), jnp.float32),
            jax.ShapeDtypeStruct((n, 1), jnp.float32),
            jax.ShapeDtypeStruct((n, 1), jnp.float32),
            jax.ShapeDtypeStruct((1, 1), jnp.float32),
            jax.ShapeDtypeStruct((1, 1), jnp.float32),
        ],
    )


_PREP_J0 = _make_prep(N_JOINT, 2000, True, 6)
_PREP_B0 = _make_prep(N_BAR, 2000, True, 4)
_PREP_J = _make_prep(N_JOINT, 2000, False, 0)
_PREP_B = _make_prep(N_BAR, 2000, False, 0)


_HBLK = 2000


def _head_body(acc, s, bias, aw1, ab1, ag, abe, aw2, ab2, batch,
               cw1, cb1, cg, cbe, cw2, cb2,
               probs_o, value_o, pooled_sc, cnt_sc):
    i = pl.program_id(0)
    b = jax.nn.relu(acc[...] / (s[...] + 1e-16) + bias[...])
    h = b @ aw1[...] + ab1[...]
    m = jnp.mean(h, axis=-1, keepdims=True)
    v = jnp.mean((h - m) ** 2, axis=-1, keepdims=True)
    h = (h - m) / jnp.sqrt(v + 1e-5) * ag[...] + abe[...]
    h = jax.nn.relu(h)
    probs_o[...] = jax.nn.sigmoid(h @ aw2[...] + ab2[...])

    @pl.when(i == 0)
    def _():
        pooled_sc[...] = jnp.zeros_like(pooled_sc)
        cnt_sc[...] = jnp.zeros_like(cnt_sc)

    gids = lax.broadcasted_iota(jnp.int32, (NUM_GRAPHS, 1), 0)
    onehot = (batch[...][:, 0][None, :] == gids).astype(jnp.float32)
    pooled_sc[...] = pooled_sc[...] + onehot @ b
    cnt_sc[...] = cnt_sc[...] + jnp.sum(onehot, axis=1, keepdims=True)

    @pl.when(i == (N_BAR // _HBLK) - 1)
    def _():
        pooled = pooled_sc[...] / jnp.maximum(cnt_sc[...], 1.0)
        h2 = pooled @ cw1[...] + cb1[...]
        m2 = jnp.mean(h2, axis=-1, keepdims=True)
        v2 = jnp.mean((h2 - m2) ** 2, axis=-1, keepdims=True)
        h2 = (h2 - m2) / jnp.sqrt(v2 + 1e-5) * cg[...] + cbe[...]
        h2 = jax.nn.relu(h2)
        value_o[...] = jnp.tanh(h2 @ cw2[...] + cb2[...])[:, 0]


def _make_head():
    vec = pl.BlockSpec((HIDDEN,), lambda i: (0,))
    mat = pl.BlockSpec((HIDDEN, HIDDEN), lambda i: (0, 0))
    col = pl.BlockSpec((HIDDEN, 1), lambda i: (0, 0))
    one = pl.BlockSpec((1,), lambda i: (0,))
    return pl.pallas_call(
        _head_body,
        grid=(N_BAR // _HBLK,),
        in_specs=[
            pl.BlockSpec((_HBLK, HIDDEN), lambda i: (i, 0)),
            pl.BlockSpec((_HBLK, 1), lambda i: (i, 0)),
            vec,
            mat, vec, vec, vec, col, one,
            pl.BlockSpec((_HBLK, 1), lambda i: (i, 0)),
            mat, vec, vec, vec, col, one,
        ],
        out_specs=[
            pl.BlockSpec((_HBLK, 1), lambda i: (i, 0)),
            pl.BlockSpec((NUM_GRAPHS,), lambda i: (0,)),
        ],
        out_shape=[
            jax.ShapeDtypeStruct((N_BAR, 1), jnp.float32),
            jax.ShapeDtypeStruct((NUM_GRAPHS,), jnp.float32),
        ],
        scratch_shapes=[
            pltpu.VMEM((NUM_GRAPHS, HIDDEN), jnp.float32),
            pltpu.VMEM((NUM_GRAPHS, 1), jnp.float32),
        ],
    )


_HEAD = _make_head()


def _gvec(ms, md):
    g = jax.nn.leaky_relu(ms[0, 0] + md[0, 0], negative_slope=0.2)
    return jnp.full((16,), g, jnp.float32)


def kernel(joint_x, bar_x, params, bj_src, bj_dst, jb_src, jb_dst, bar_batch):
    p = params
    cnt_bj = _COUNT[2](bj_dst)
    srcb_bj, dstb_bj, offs_bj = _PLACE[2](bj_src, bj_dst, cnt_bj)
    cnt_jb = _COUNT[16](jb_dst)
    srcb_jb, dstb_jb, offs_jb = _PLACE[16](jb_src, jb_dst, cnt_jb)

    acc_j = s_j = acc_b = s_b = None
    for l in range(NUM_LAYERS):
        lp = p['layers'][l]
        if l == 0:
            xs_b, a_sb, a_db, msb, mdb = _PREP_B0(
                bar_x, p['be_W'], p['be_b'],
                lp['bj']['W'], lp['bj']['att_src'],
                lp['jb']['W'], lp['jb']['att_dst'])
            xs_j, a_sj, a_dj, msj, mdj = _PREP_J0(
                joint_x, p['je_W'], p['je_b'],
                lp['jb']['W'], lp['jb']['att_src'],
                lp['bj']['W'], lp['bj']['att_dst'])
        else:
            lpp = p['layers'][l - 1]
            xs_b, a_sb, a_db, msb, mdb = _PREP_B(
                acc_b, s_b, lpp['jb']['bias'],
                lp['bj']['W'], lp['bj']['att_src'],
                lp['jb']['W'], lp['jb']['att_dst'])
            xs_j, a_sj, a_dj, msj, mdj = _PREP_J(
                acc_j, s_j, lpp['bj']['bias'],
                lp['jb']['W'], lp['jb']['att_src'],
                lp['bj']['W'], lp['bj']['att_dst'])

        acc_j, s_j = _AGG_BJ(srcb_bj, dstb_bj, offs_bj, a_sb[:, 0], a_dj[:, 0],
                             xs_b, _gvec(msb, mdj))
        acc_b, s_b = _AGG_JB(srcb_jb, dstb_jb, offs_jb, a_sj[:, 0], a_db[:, 0],
                             xs_j, _gvec(msj, mdb))
        s_j = s_j[:, None]
        s_b = s_b[:, None]

    probs, value = _HEAD(
        acc_b, s_b, p['layers'][NUM_LAYERS - 1]['jb']['bias'],
        p['a_W1'], p['a_b1'], p['a_g'], p['a_be'], p['a_W2'], p['a_b2'],
        bar_batch[:, None],
        p['c_W1'], p['c_b1'], p['c_g'], p['c_be'], p['c_W2'], p['c_b2'])
    return probs[:, 0], value
